# baseline (speedup 1.0000x reference)
import functools

import jax
import jax.numpy as jnp
from jax import lax
from jax.experimental import pallas as pl
from jax.experimental.pallas import tpu as pltpu
from jax.experimental.pallas import tpu_sc as plsc

NUM_ENT = 50000
NUM_REL = 500
REL_PAD = 512
D = 64
DP = 128
K = 5
B = 16384
NC = 2
NS = 16
NW = NC * NS
CHUNK = B // NW
JCH = 128
NJ = CHUNK // JCH
TB = 8192
NTB = -(-NUM_ENT // TB)
ENT_PAD = NTB * TB


def _tr_body(in_ref, out_ref):
    eye = jnp.pad(jnp.eye(D, dtype=jnp.float32), ((0, 0), (0, DP - D)))
    out_ref[...] = jax.lax.dot_general(
        in_ref[...], eye,
        dimension_numbers=(((0,), (0,)), ((), ())),
        preferred_element_type=jnp.float32)


_tr_call = pl.pallas_call(
    _tr_body,
    grid=(NTB,),
    in_specs=[pl.BlockSpec((D, TB), lambda p: (0, p))],
    out_specs=pl.BlockSpec((TB, DP), lambda p: (p, 0)),
    out_shape=jax.ShapeDtypeStruct((ENT_PAD, DP), jnp.float32),
)


def _mask_body(rel_ref, w_ref, b_ref, out_ref):
    rel = rel_ref[...]
    w = w_ref[...]
    bias = b_ref[0, 0]
    s = jnp.sum(rel * w, axis=1, keepdims=True) + bias
    rid = lax.broadcasted_iota(jnp.int32, (REL_PAD, 1), 0)
    neg = jnp.float32(-jnp.inf)
    s = jnp.where(rid < NUM_REL, s, neg)
    mask = jnp.zeros((REL_PAD, 1), jnp.float32)
    rem = s
    for _ in range(K):
        cur = jnp.max(rem)
        ismax = rem == cur
        first = jnp.min(jnp.where(ismax, rid, jnp.int32(1 << 30)))
        sel = rid == first
        mask = jnp.where(sel, jnp.float32(1.0), mask)
        rem = jnp.where(sel, neg, rem)
    out_ref[...] = jnp.concatenate(
        [rel * mask, jnp.zeros((REL_PAD, DP - D), jnp.float32)], axis=1)


_mask_call = pl.pallas_call(
    _mask_body,
    out_shape=jax.ShapeDtypeStruct((REL_PAD, DP), jnp.float32),
)


def _sc_body(ent_hbm, relm_hbm, h_hbm, r_hbm, t_hbm, out_hbm,
             idx_h, idx_r, idx_t,
             hv0, rv0, tv0, hv1, rv1, tv1, hv2, rv2, tv2, hv3, rv3, tv3,
             pt, outv, s0, s1, s2, s3):
    wid = lax.axis_index("s") * NC + lax.axis_index("c")
    base = wid * CHUNK

    ic = (
        pltpu.async_copy(h_hbm.at[pl.ds(base, CHUNK)], idx_h, s0),
        pltpu.async_copy(r_hbm.at[pl.ds(base, CHUNK)], idx_r, s0),
        pltpu.async_copy(t_hbm.at[pl.ds(base, CHUNK)], idx_t, s0),
    )
    for cp in ic:
        cp.wait()

    bufs = ((hv0, rv0, tv0, s0), (hv1, rv1, tv1, s1),
            (hv2, rv2, tv2, s2), (hv3, rv3, tv3, s3))
    cps = {}

    def issue(j):
        hb, rb, tb, sem = bufs[j]
        sl = pl.ds(j * JCH, JCH)
        cps[j] = (
            pltpu.async_copy(ent_hbm.at[idx_h.at[sl]], hb, sem),
            pltpu.async_copy(relm_hbm.at[idx_r.at[sl]], rb, sem),
            pltpu.async_copy(ent_hbm.at[idx_t.at[sl]], tb, sem),
        )

    for j in range(NJ):
        issue(j)

    lanes = lax.iota(jnp.int32, 16)
    pidx = lanes * CHUNK

    for j in range(NJ):
        for cp in cps[j]:
            cp.wait()
        hb, rb, tb, _ = bufs[j]

        @plsc.parallel_loop(0, JCH, unroll=8)
        def row_body(i, hb=hb, rb=rb, tb=tb, j=j):
            acc = jnp.zeros((16,), jnp.float32)
            for c in range(D // 16):
                sl = pl.ds(c * 16, 16)
                acc = acc + (hb[i, sl] + rb[i, sl]) * tb[i, sl]
            plsc.store_scatter(pt, [pidx + (j * JCH + i)], acc)

    @plsc.parallel_loop(0, CHUNK // 16, unroll=4)
    def red_body(q):
        vec = jnp.zeros((16,), jnp.float32)
        for d in range(16):
            vec = vec + pt[pl.ds(d * CHUNK + q * 16, 16)]
        outv[pl.ds(q * 16, 16)] = vec

    pltpu.sync_copy(outv, out_hbm.at[pl.ds(base, CHUNK)])


_sc_call = functools.partial(
    pl.kernel,
    mesh=plsc.VectorSubcoreMesh(core_axis_name="c", subcore_axis_name="s"),
    out_type=jax.ShapeDtypeStruct((B,), jnp.float32),
    compiler_params=pltpu.CompilerParams(
        needs_layout_passes=False, use_tc_tiling_on_sc=False),
    scratch_types=[
        pltpu.VMEM((CHUNK,), jnp.int32),
        pltpu.VMEM((CHUNK,), jnp.int32),
        pltpu.VMEM((CHUNK,), jnp.int32),
        pltpu.VMEM((JCH, D), jnp.float32),
        pltpu.VMEM((JCH, D), jnp.float32),
        pltpu.VMEM((JCH, D), jnp.float32),
        pltpu.VMEM((JCH, D), jnp.float32),
        pltpu.VMEM((JCH, D), jnp.float32),
        pltpu.VMEM((JCH, D), jnp.float32),
        pltpu.VMEM((JCH, D), jnp.float32),
        pltpu.VMEM((JCH, D), jnp.float32),
        pltpu.VMEM((JCH, D), jnp.float32),
        pltpu.VMEM((JCH, D), jnp.float32),
        pltpu.VMEM((JCH, D), jnp.float32),
        pltpu.VMEM((JCH, D), jnp.float32),
        pltpu.VMEM((16 * CHUNK,), jnp.float32),
        pltpu.VMEM((CHUNK,), jnp.float32),
        pltpu.SemaphoreType.DMA,
        pltpu.SemaphoreType.DMA,
        pltpu.SemaphoreType.DMA,
        pltpu.SemaphoreType.DMA,
    ],
)(_sc_body)


def kernel(entity_emb, relation_emb, sel_w, sel_b, lin_w, lin_b, h, r, t, edge_index):
    entv = _tr_call(entity_emb.T).reshape(2 * ENT_PAD, D)
    relp = jnp.pad(relation_emb, ((0, REL_PAD - NUM_REL), (0, 0)))
    relm = _mask_call(relp, sel_w.reshape(1, D), sel_b.reshape(1, 1))
    relv = relm.reshape(2 * REL_PAD, D)
    return _sc_call(entv, relv, h + h, r + r, t + t)


# speedup vs baseline: 1.0736x; 1.0736x over previous
import functools

import jax
import jax.numpy as jnp
from jax import lax
from jax.experimental import pallas as pl
from jax.experimental.pallas import tpu as pltpu
from jax.experimental.pallas import tpu_sc as plsc

NUM_ENT = 50000
NUM_REL = 500
REL_PAD = 512
D = 64
DP = 128
K = 5
B = 16384
NC = 2
NS = 16
NW = NC * NS
CHUNK = B // NW
JCH = 128
NJ = CHUNK // JCH
TB = 8192
NTB = -(-NUM_ENT // TB)
ENT_PAD = NTB * TB


def _tr_body(in_ref, rel_ref, w_ref, b_ref, out_ref, relm_ref):
    eye = jnp.pad(jnp.eye(D, dtype=jnp.float32), ((0, 0), (0, DP - D)))
    out_ref[...] = jax.lax.dot_general(
        in_ref[...], eye,
        dimension_numbers=(((0,), (0,)), ((), ())),
        preferred_element_type=jnp.float32)

    @pl.when(pl.program_id(0) == 0)
    def _():
        rel = rel_ref[...]
        w = w_ref[...]
        bias = b_ref[0, 0]
        s = jnp.sum(rel * w, axis=1, keepdims=True) + bias
        rid = lax.broadcasted_iota(jnp.int32, (REL_PAD, 1), 0)
        neg = jnp.float32(-jnp.inf)
        s = jnp.where(rid < NUM_REL, s, neg)
        mask = jnp.zeros((REL_PAD, 1), jnp.float32)
        rem = s
        for _ in range(K):
            cur = jnp.max(rem)
            ismax = rem == cur
            first = jnp.min(jnp.where(ismax, rid, jnp.int32(1 << 30)))
            sel = rid == first
            mask = jnp.where(sel, jnp.float32(1.0), mask)
            rem = jnp.where(sel, neg, rem)
        relm_ref[...] = jnp.concatenate(
            [rel * mask, jnp.zeros((REL_PAD, DP - D), jnp.float32)], axis=1)


_tr_call = pl.pallas_call(
    _tr_body,
    grid=(NTB,),
    in_specs=[
        pl.BlockSpec((D, TB), lambda p: (0, p)),
        pl.BlockSpec((REL_PAD, D), lambda p: (0, 0)),
        pl.BlockSpec((1, D), lambda p: (0, 0)),
        pl.BlockSpec((1, 1), lambda p: (0, 0)),
    ],
    out_specs=[
        pl.BlockSpec((TB, DP), lambda p: (p, 0)),
        pl.BlockSpec((REL_PAD, DP), lambda p: (0, 0)),
    ],
    out_shape=[
        jax.ShapeDtypeStruct((ENT_PAD, DP), jnp.float32),
        jax.ShapeDtypeStruct((REL_PAD, DP), jnp.float32),
    ],
)


def _sc_body(ent_hbm, relm_hbm, h_hbm, r_hbm, t_hbm, out_hbm,
             idx_h, idx_r, idx_t,
             hv0, rv0, tv0, hv1, rv1, tv1, hv2, rv2, tv2, hv3, rv3, tv3,
             pt, outv, s0, s1, s2, s3):
    wid = lax.axis_index("s") * NC + lax.axis_index("c")
    base = wid * CHUNK

    ic = (
        pltpu.async_copy(h_hbm.at[pl.ds(base, CHUNK)], idx_h, s0),
        pltpu.async_copy(r_hbm.at[pl.ds(base, CHUNK)], idx_r, s0),
        pltpu.async_copy(t_hbm.at[pl.ds(base, CHUNK)], idx_t, s0),
    )
    for cp in ic:
        cp.wait()

    bufs = ((hv0, rv0, tv0, s0), (hv1, rv1, tv1, s1),
            (hv2, rv2, tv2, s2), (hv3, rv3, tv3, s3))
    cps = {}

    def issue(j):
        hb, rb, tb, sem = bufs[j]
        sl = pl.ds(j * JCH, JCH)
        cps[j] = (
            pltpu.async_copy(ent_hbm.at[idx_h.at[sl]], hb, sem),
            pltpu.async_copy(relm_hbm.at[idx_r.at[sl]], rb, sem),
            pltpu.async_copy(ent_hbm.at[idx_t.at[sl]], tb, sem),
        )

    issue(0)
    issue(1)

    lanes = lax.iota(jnp.int32, 16)
    pidx = lanes * CHUNK

    for j in range(NJ):
        for cp in cps[j]:
            cp.wait()
        hb, rb, tb, _ = bufs[j]

        @plsc.parallel_loop(0, JCH, unroll=8)
        def row_body(i, hb=hb, rb=rb, tb=tb, j=j):
            acc = jnp.zeros((16,), jnp.float32)
            for c in range(D // 16):
                sl = pl.ds(c * 16, 16)
                acc = acc + (hb[i, sl] + rb[i, sl]) * tb[i, sl]
            plsc.store_scatter(pt, [pidx + (j * JCH + i)], acc)

        if j + 2 < NJ:
            issue(j + 2)

    @plsc.parallel_loop(0, CHUNK // 16, unroll=4)
    def red_body(q):
        vec = jnp.zeros((16,), jnp.float32)
        for d in range(16):
            vec = vec + pt[pl.ds(d * CHUNK + q * 16, 16)]
        outv[pl.ds(q * 16, 16)] = vec

    pltpu.sync_copy(outv, out_hbm.at[pl.ds(base, CHUNK)])


_sc_call = functools.partial(
    pl.kernel,
    mesh=plsc.VectorSubcoreMesh(core_axis_name="c", subcore_axis_name="s"),
    out_type=jax.ShapeDtypeStruct((B,), jnp.float32),
    compiler_params=pltpu.CompilerParams(
        needs_layout_passes=False, use_tc_tiling_on_sc=False),
    scratch_types=[
        pltpu.VMEM((CHUNK,), jnp.int32),
        pltpu.VMEM((CHUNK,), jnp.int32),
        pltpu.VMEM((CHUNK,), jnp.int32),
        pltpu.VMEM((JCH, D), jnp.float32),
        pltpu.VMEM((JCH, D), jnp.float32),
        pltpu.VMEM((JCH, D), jnp.float32),
        pltpu.VMEM((JCH, D), jnp.float32),
        pltpu.VMEM((JCH, D), jnp.float32),
        pltpu.VMEM((JCH, D), jnp.float32),
        pltpu.VMEM((JCH, D), jnp.float32),
        pltpu.VMEM((JCH, D), jnp.float32),
        pltpu.VMEM((JCH, D), jnp.float32),
        pltpu.VMEM((JCH, D), jnp.float32),
        pltpu.VMEM((JCH, D), jnp.float32),
        pltpu.VMEM((JCH, D), jnp.float32),
        pltpu.VMEM((16 * CHUNK,), jnp.float32),
        pltpu.VMEM((CHUNK,), jnp.float32),
        pltpu.SemaphoreType.DMA,
        pltpu.SemaphoreType.DMA,
        pltpu.SemaphoreType.DMA,
        pltpu.SemaphoreType.DMA,
    ],
)(_sc_body)


def kernel(entity_emb, relation_emb, sel_w, sel_b, lin_w, lin_b, h, r, t, edge_index):
    relp = jnp.pad(relation_emb, ((0, REL_PAD - NUM_REL), (0, 0)))
    entp, relm = _tr_call(
        entity_emb.T, relp, sel_w.reshape(1, D), sel_b.reshape(1, 1))
    entv = entp.reshape(2 * ENT_PAD, D)
    relv = relm.reshape(2 * REL_PAD, D)
    return _sc_call(entv, relv, h + h, r + r, t + t)


# speedup vs baseline: 1.0792x; 1.0052x over previous
import functools

import jax
import jax.numpy as jnp
from jax import lax
from jax.experimental import pallas as pl
from jax.experimental.pallas import tpu as pltpu
from jax.experimental.pallas import tpu_sc as plsc

NUM_ENT = 50000
NUM_REL = 500
REL_PAD = 512
D = 64
DP = 128
K = 5
B = 16384
NC = 2
NS = 16
NW = NC * NS
CHUNK = B // NW
JCH = 128
NJ = CHUNK // JCH
TB = 7168
NTB = -(-NUM_ENT // TB)
ENT_PAD = NTB * TB


def _tr_body(in_ref, rel_ref, w_ref, b_ref, out_ref, relm_ref):
    eye = jnp.pad(jnp.eye(D, dtype=jnp.float32), ((0, 0), (0, DP - D)))
    out_ref[...] = jax.lax.dot_general(
        in_ref[...], eye,
        dimension_numbers=(((0,), (0,)), ((), ())),
        preferred_element_type=jnp.float32)

    @pl.when(pl.program_id(0) == 0)
    def _():
        rel = rel_ref[...]
        w = w_ref[...]
        bias = b_ref[0, 0]
        s = jnp.sum(rel * w, axis=1, keepdims=True) + bias
        rid = lax.broadcasted_iota(jnp.int32, (REL_PAD, 1), 0)
        neg = jnp.float32(-jnp.inf)
        s = jnp.where(rid < NUM_REL, s, neg)
        mask = jnp.zeros((REL_PAD, 1), jnp.float32)
        rem = s
        for _ in range(K):
            cur = jnp.max(rem)
            ismax = rem == cur
            first = jnp.min(jnp.where(ismax, rid, jnp.int32(1 << 30)))
            sel = rid == first
            mask = jnp.where(sel, jnp.float32(1.0), mask)
            rem = jnp.where(sel, neg, rem)
        relm_ref[...] = jnp.concatenate(
            [rel * mask, jnp.zeros((REL_PAD, DP - D), jnp.float32)], axis=1)


_tr_call = pl.pallas_call(
    _tr_body,
    grid=(NTB,),
    in_specs=[
        pl.BlockSpec((D, TB), lambda p: (0, p)),
        pl.BlockSpec((REL_PAD, D), lambda p: (0, 0)),
        pl.BlockSpec((1, D), lambda p: (0, 0)),
        pl.BlockSpec((1, 1), lambda p: (0, 0)),
    ],
    out_specs=[
        pl.BlockSpec((TB, DP), lambda p: (p, 0)),
        pl.BlockSpec((REL_PAD, DP), lambda p: (0, 0)),
    ],
    out_shape=[
        jax.ShapeDtypeStruct((ENT_PAD, DP), jnp.float32),
        jax.ShapeDtypeStruct((REL_PAD, DP), jnp.float32),
    ],
)


def _sc_body(ent_hbm, relm_hbm, h_hbm, r_hbm, t_hbm, out_hbm,
             idx_h, idx_r, idx_t,
             hv0, rv0, tv0, hv1, rv1, tv1, hv2, rv2, tv2, hv3, rv3, tv3,
             pt, outv, s0, s1, s2, s3):
    wid = lax.axis_index("s") * NC + lax.axis_index("c")
    base = wid * CHUNK

    ic = (
        pltpu.async_copy(h_hbm.at[pl.ds(base, CHUNK)], idx_h, s0),
        pltpu.async_copy(r_hbm.at[pl.ds(base, CHUNK)], idx_r, s0),
        pltpu.async_copy(t_hbm.at[pl.ds(base, CHUNK)], idx_t, s0),
    )
    for cp in ic:
        cp.wait()

    bufs = ((hv0, rv0, tv0, s0), (hv1, rv1, tv1, s1),
            (hv2, rv2, tv2, s2), (hv3, rv3, tv3, s3))
    cps = {}

    def issue(j):
        hb, rb, tb, sem = bufs[j]
        sl = pl.ds(j * JCH, JCH)
        cps[j] = (
            pltpu.async_copy(ent_hbm.at[idx_h.at[sl]], hb, sem),
            pltpu.async_copy(relm_hbm.at[idx_r.at[sl]], rb, sem),
            pltpu.async_copy(ent_hbm.at[idx_t.at[sl]], tb, sem),
        )

    issue(0)
    issue(1)

    lanes = lax.iota(jnp.int32, 16)
    pidx = lanes * CHUNK

    for j in range(NJ):
        for cp in cps[j]:
            cp.wait()
        hb, rb, tb, _ = bufs[j]

        @plsc.parallel_loop(0, JCH, unroll=8)
        def row_body(i, hb=hb, rb=rb, tb=tb, j=j):
            acc = jnp.zeros((16,), jnp.float32)
            for c in range(D // 16):
                sl = pl.ds(c * 16, 16)
                acc = acc + (hb[i, sl] + rb[i, sl]) * tb[i, sl]
            plsc.store_scatter(pt, [pidx + (j * JCH + i)], acc)

        if j + 2 < NJ:
            issue(j + 2)

    @plsc.parallel_loop(0, CHUNK // 16, unroll=4)
    def red_body(q):
        vec = jnp.zeros((16,), jnp.float32)
        for d in range(16):
            vec = vec + pt[pl.ds(d * CHUNK + q * 16, 16)]
        outv[pl.ds(q * 16, 16)] = vec

    pltpu.sync_copy(outv, out_hbm.at[pl.ds(base, CHUNK)])


_sc_call = functools.partial(
    pl.kernel,
    mesh=plsc.VectorSubcoreMesh(core_axis_name="c", subcore_axis_name="s"),
    out_type=jax.ShapeDtypeStruct((B,), jnp.float32),
    compiler_params=pltpu.CompilerParams(
        needs_layout_passes=False, use_tc_tiling_on_sc=False),
    scratch_types=[
        pltpu.VMEM((CHUNK,), jnp.int32),
        pltpu.VMEM((CHUNK,), jnp.int32),
        pltpu.VMEM((CHUNK,), jnp.int32),
        pltpu.VMEM((JCH, D), jnp.float32),
        pltpu.VMEM((JCH, D), jnp.float32),
        pltpu.VMEM((JCH, D), jnp.float32),
        pltpu.VMEM((JCH, D), jnp.float32),
        pltpu.VMEM((JCH, D), jnp.float32),
        pltpu.VMEM((JCH, D), jnp.float32),
        pltpu.VMEM((JCH, D), jnp.float32),
        pltpu.VMEM((JCH, D), jnp.float32),
        pltpu.VMEM((JCH, D), jnp.float32),
        pltpu.VMEM((JCH, D), jnp.float32),
        pltpu.VMEM((JCH, D), jnp.float32),
        pltpu.VMEM((JCH, D), jnp.float32),
        pltpu.VMEM((16 * CHUNK,), jnp.float32),
        pltpu.VMEM((CHUNK,), jnp.float32),
        pltpu.SemaphoreType.DMA,
        pltpu.SemaphoreType.DMA,
        pltpu.SemaphoreType.DMA,
        pltpu.SemaphoreType.DMA,
    ],
)(_sc_body)


def kernel(entity_emb, relation_emb, sel_w, sel_b, lin_w, lin_b, h, r, t, edge_index):
    relp = jnp.pad(relation_emb, ((0, REL_PAD - NUM_REL), (0, 0)))
    entp, relm = _tr_call(
        entity_emb.T, relp, sel_w.reshape(1, D), sel_b.reshape(1, 1))
    entv = entp.reshape(2 * ENT_PAD, D)
    relv = relm.reshape(2 * REL_PAD, D)
    return _sc_call(entv, relv, h + h, r + r, t + t)
